# baseline (speedup 1.0000x reference)
import functools

import jax
import jax.numpy as jnp
from jax import lax
from jax.experimental import pallas as pl
from jax.experimental.pallas import tpu as pltpu
from jax.experimental.pallas import tpu_sc as plsc

N1 = 100000
D = 128
N2 = 25000
MAX_NUM = 16

NC = 2
NS = 16
NW = NC * NS

B = 8
IDX_PER_CHUNK = B * MAX_NUM
NCHUNKS = N2 // B
CPW = (NCHUNKS + NW - 1) // NW
BASE = NCHUNKS // NW
EXTRA = NCHUNKS - BASE * NW
SPLIT = NW - EXTRA

SHADOW_BLK = 5000


def _shadow_body(x_ref, o_ref):
    i = pl.program_id(0)
    m = jnp.min(x_ref[...], axis=0, keepdims=True)

    @pl.when(i == 0)
    def _init():
        o_ref[...] = m

    @pl.when(i != 0)
    def _acc():
        o_ref[...] = jnp.minimum(o_ref[...], m)


def _shadow_row(x):
    return pl.pallas_call(
        _shadow_body,
        grid=(N1 // SHADOW_BLK,),
        in_specs=[pl.BlockSpec((SHADOW_BLK, D), lambda i: (i, 0))],
        out_specs=pl.BlockSpec((1, D), lambda i: (0, 0)),
        out_shape=jax.ShapeDtypeStruct((1, D), jnp.float32),
    )(x)


def _lane_max(v):
    iota = lax.iota(jnp.int32, 16)
    for k in (1, 2, 4, 8):
        perm = iota ^ k
        v = jnp.maximum(v, v.at[perm].get(mode="promise_in_bounds"))
    return v


def _pool_body(x_hbm, inds_hbm, shadow_hbm, out_hbm,
               idxs_v, idxg0, idxg1, idxg2, rows0, rows1, rows2,
               outb0, outb1, outb2, flag0, flag1, flag2, shv_v,
               sem0, sem1, sem2, osem0, osem1, osem2):
    w = lax.axis_index("s") * NC + lax.axis_index("c")
    start = w * BASE + jnp.maximum(w - SPLIT, 0)
    count = BASE + jnp.where(w >= SPLIT, 1, 0)

    pltpu.sync_copy(shadow_hbm, shv_v)
    pltpu.sync_copy(
        inds_hbm.at[pl.ds(start * IDX_PER_CHUNK, CPW * IDX_PER_CHUNK)],
        idxs_v)

    idxg = (idxg0, idxg1, idxg2)
    rows = (rows0, rows1, rows2)
    outb = (outb0, outb1, outb2)
    flag = (flag0, flag1, flag2)
    sems = (sem0, sem1, sem2)
    osems = (osem0, osem1, osem2)

    def stage(i, b):
        @pl.when(i < count)
        def _():
            off = i * IDX_PER_CHUNK
            m = idxs_v[pl.ds(off, 16)]
            for r in range(1, B):
                m = jnp.maximum(m, idxs_v[pl.ds(off + r * MAX_NUM, 16)])
            dirty = jnp.where(_lane_max(m) >= N1, 1, 0)
            flag[b][0] = dirty[0]

            @pl.when(flag[b][0] == 0)
            def _clean():
                pltpu.async_copy(
                    x_hbm.at[idxs_v.at[pl.ds(off, IDX_PER_CHUNK)]],
                    rows[b], sems[b])

            @pl.when(flag[b][0] != 0)
            def _dirty():
                def pre(r, c):
                    iv = idxs_v[pl.ds(off + r * MAX_NUM, MAX_NUM)]
                    valid = iv < N1
                    fb = jnp.maximum(_lane_max(jnp.where(valid, iv, -1)), 0)
                    idxg[b][pl.ds(r * MAX_NUM, MAX_NUM)] = (
                        jnp.where(valid, iv, fb))
                    return c

                lax.fori_loop(0, B, pre, 0, unroll=True)
                pltpu.async_copy(x_hbm.at[idxg[b]], rows[b], sems[b])

    def consume(i, b):
        @pl.when(i < count)
        def _():
            pltpu.make_async_copy(x_hbm.at[idxg[b]], rows[b], sems[b]).wait()

            @pl.when(i >= 3)
            def _drain_prev():
                pltpu.make_async_copy(
                    outb[b], out_hbm.at[pl.ds(0, B), :], osems[b]).wait()

            @pl.when(flag[b][0] == 0)
            def _clean():
                def comp(r, c):
                    base = r * MAX_NUM
                    for col in range(D // 16):
                        acc = rows[b][base, pl.ds(col * 16, 16)]
                        for j in range(1, MAX_NUM):
                            acc = jnp.maximum(
                                acc, rows[b][base + j, pl.ds(col * 16, 16)])
                        outb[b][r, pl.ds(col * 16, 16)] = acc
                    return c

                lax.fori_loop(0, B, comp, 0)

            @pl.when(flag[b][0] != 0)
            def _dirty():
                def comp(r, c):
                    iv = idxs_v[pl.ds(i * IDX_PER_CHUNK + r * MAX_NUM,
                                      MAX_NUM)]
                    anyv = _lane_max(jnp.where(iv < N1, iv, -1)) >= 0
                    base = r * MAX_NUM
                    for col in range(D // 16):
                        acc = rows[b][base, pl.ds(col * 16, 16)]
                        for j in range(1, MAX_NUM):
                            acc = jnp.maximum(
                                acc, rows[b][base + j, pl.ds(col * 16, 16)])
                        sh = shv_v[pl.ds(col * 16, 16)]
                        outb[b][r, pl.ds(col * 16, 16)] = (
                            jnp.where(anyv, acc, sh))
                    return c

                lax.fori_loop(0, B, comp, 0)

            pltpu.async_copy(
                outb[b], out_hbm.at[pl.ds((start + i) * B, B), :], osems[b])

    stage(0, 0)
    stage(1, 1)

    def outer(t, carry):
        i0 = t * 3
        for b_off in range(3):
            i = i0 + b_off
            consume(i, b_off)
            stage(i + 2, (b_off + 2) % 3)
        return carry

    lax.fori_loop(0, (CPW + 2) // 3, outer, 0)

    for b in (0, 1, 2):
        pltpu.make_async_copy(
            outb[b], out_hbm.at[pl.ds(0, B), :], osems[b]).wait()


def _pool(x, inds_flat, shadow):
    mesh = plsc.VectorSubcoreMesh(core_axis_name="c", subcore_axis_name="s")
    return pl.kernel(
        _pool_body,
        out_type=jax.ShapeDtypeStruct((N2, D), jnp.float32),
        mesh=mesh,
        scratch_types=[
            pltpu.VMEM((CPW * IDX_PER_CHUNK,), jnp.int32),
            pltpu.VMEM((IDX_PER_CHUNK,), jnp.int32),
            pltpu.VMEM((IDX_PER_CHUNK,), jnp.int32),
            pltpu.VMEM((IDX_PER_CHUNK,), jnp.int32),
            pltpu.VMEM((IDX_PER_CHUNK, D), jnp.float32),
            pltpu.VMEM((IDX_PER_CHUNK, D), jnp.float32),
            pltpu.VMEM((IDX_PER_CHUNK, D), jnp.float32),
            pltpu.VMEM((B, D), jnp.float32),
            pltpu.VMEM((B, D), jnp.float32),
            pltpu.VMEM((B, D), jnp.float32),
            pltpu.SMEM((1,), jnp.int32),
            pltpu.SMEM((1,), jnp.int32),
            pltpu.SMEM((1,), jnp.int32),
            pltpu.VMEM((D,), jnp.float32),
            pltpu.SemaphoreType.DMA,
            pltpu.SemaphoreType.DMA,
            pltpu.SemaphoreType.DMA,
            pltpu.SemaphoreType.DMA,
            pltpu.SemaphoreType.DMA,
            pltpu.SemaphoreType.DMA,
        ],
    )(x, inds_flat, shadow)


def kernel(x, inds):
    inds_flat = inds.astype(jnp.int32).reshape(-1)
    shadow = _shadow_row(x).reshape(-1)
    return _pool(x, inds_flat, shadow)


# speedup vs baseline: 1.1698x; 1.1698x over previous
import functools

import jax
import jax.numpy as jnp
from jax import lax
from jax.experimental import pallas as pl
from jax.experimental.pallas import tpu as pltpu
from jax.experimental.pallas import tpu_sc as plsc

N1 = 100000
D = 128
N2 = 25000
MAX_NUM = 16

NC = 2
NS = 16
NW = NC * NS

B = 8
IDX_PER_CHUNK = B * MAX_NUM
NCHUNKS = N2 // B
CPW = (NCHUNKS + NW - 1) // NW
BASE = NCHUNKS // NW
EXTRA = NCHUNKS - BASE * NW
SPLIT = NW - EXTRA

SHADOW_BLK = 5000


def _shadow_body(x_ref, o_ref):
    i = pl.program_id(0)
    m = jnp.min(x_ref[...], axis=0, keepdims=True)

    @pl.when(i == 0)
    def _init():
        o_ref[...] = m

    @pl.when(i != 0)
    def _acc():
        o_ref[...] = jnp.minimum(o_ref[...], m)


def _shadow_row(x):
    return pl.pallas_call(
        _shadow_body,
        grid=(N1 // SHADOW_BLK,),
        in_specs=[pl.BlockSpec((SHADOW_BLK, D), lambda i: (i, 0))],
        out_specs=pl.BlockSpec((1, D), lambda i: (0, 0)),
        out_shape=jax.ShapeDtypeStruct((1, D), jnp.float32),
    )(x)


def _lane_max(v):
    iota = lax.iota(jnp.int32, 16)
    for k in (1, 2, 4, 8):
        perm = iota ^ k
        v = jnp.maximum(v, v.at[perm].get(mode="promise_in_bounds"))
    return v


def _pool_body(x_hbm, inds_hbm, shadow_hbm, out_hbm,
               idxs_v, idxg0, idxg1, rows0, rows1, outb0, outb1,
               flag0, flag1, shv_v,
               sem0, sem1, osem0, osem1):
    w = lax.axis_index("s") * NC + lax.axis_index("c")
    start = w * BASE + jnp.maximum(w - SPLIT, 0)
    count = BASE + jnp.where(w >= SPLIT, 1, 0)

    pltpu.sync_copy(shadow_hbm, shv_v)
    pltpu.sync_copy(
        inds_hbm.at[pl.ds(start * IDX_PER_CHUNK, CPW * IDX_PER_CHUNK)],
        idxs_v)

    idxg = (idxg0, idxg1)
    rows = (rows0, rows1)
    outb = (outb0, outb1)
    flag = (flag0, flag1)
    sems = (sem0, sem1)
    osems = (osem0, osem1)

    def stage(i, b):
        @pl.when(i < count)
        def _():
            off = i * IDX_PER_CHUNK
            m = idxs_v[pl.ds(off, 16)]
            for r in range(1, B):
                m = jnp.maximum(m, idxs_v[pl.ds(off + r * MAX_NUM, 16)])
            dirty = jnp.where(_lane_max(m) >= N1, 1, 0)
            flag[b][0] = dirty[0]

            @pl.when(flag[b][0] == 0)
            def _clean():
                pltpu.async_copy(
                    x_hbm.at[idxs_v.at[pl.ds(off, IDX_PER_CHUNK)]],
                    rows[b], sems[b])

            @pl.when(flag[b][0] != 0)
            def _dirty():
                def pre(r, c):
                    iv = idxs_v[pl.ds(off + r * MAX_NUM, MAX_NUM)]
                    valid = iv < N1
                    fb = jnp.maximum(_lane_max(jnp.where(valid, iv, -1)), 0)
                    idxg[b][pl.ds(r * MAX_NUM, MAX_NUM)] = (
                        jnp.where(valid, iv, fb))
                    return c

                lax.fori_loop(0, B, pre, 0, unroll=True)
                pltpu.async_copy(x_hbm.at[idxg[b]], rows[b], sems[b])

    def consume(i, b):
        @pl.when(i < count)
        def _():
            pltpu.make_async_copy(x_hbm.at[idxg[b]], rows[b], sems[b]).wait()

            @pl.when(i >= 2)
            def _drain_prev():
                pltpu.make_async_copy(
                    outb[b], out_hbm.at[pl.ds(0, B), :], osems[b]).wait()

            @pl.when(flag[b][0] == 0)
            def _clean():
                def comp(r, c):
                    base = r * MAX_NUM
                    for col in range(D // 16):
                        acc = rows[b][base, pl.ds(col * 16, 16)]
                        for j in range(1, MAX_NUM):
                            acc = jnp.maximum(
                                acc, rows[b][base + j, pl.ds(col * 16, 16)])
                        outb[b][r, pl.ds(col * 16, 16)] = acc
                    return c

                lax.fori_loop(0, B, comp, 0)

            @pl.when(flag[b][0] != 0)
            def _dirty():
                def comp(r, c):
                    iv = idxs_v[pl.ds(i * IDX_PER_CHUNK + r * MAX_NUM,
                                      MAX_NUM)]
                    anyv = _lane_max(jnp.where(iv < N1, iv, -1)) >= 0
                    base = r * MAX_NUM
                    for col in range(D // 16):
                        acc = rows[b][base, pl.ds(col * 16, 16)]
                        for j in range(1, MAX_NUM):
                            acc = jnp.maximum(
                                acc, rows[b][base + j, pl.ds(col * 16, 16)])
                        sh = shv_v[pl.ds(col * 16, 16)]
                        outb[b][r, pl.ds(col * 16, 16)] = (
                            jnp.where(anyv, acc, sh))
                    return c

                lax.fori_loop(0, B, comp, 0)

            pltpu.async_copy(
                outb[b], out_hbm.at[pl.ds((start + i) * B, B), :], osems[b])

    stage(0, 0)

    def outer(t, carry):
        i0 = t * 2
        stage(i0 + 1, 1)
        consume(i0, 0)
        stage(i0 + 2, 0)
        consume(i0 + 1, 1)
        return carry

    lax.fori_loop(0, CPW // 2, outer, 0)

    for b in (0, 1):
        pltpu.make_async_copy(
            outb[b], out_hbm.at[pl.ds(0, B), :], osems[b]).wait()


def _pool(x, inds_flat, shadow):
    mesh = plsc.VectorSubcoreMesh(core_axis_name="c", subcore_axis_name="s")
    return pl.kernel(
        _pool_body,
        out_type=jax.ShapeDtypeStruct((N2, D), jnp.float32),
        mesh=mesh,
        scratch_types=[
            pltpu.VMEM((CPW * IDX_PER_CHUNK,), jnp.int32),
            pltpu.VMEM((IDX_PER_CHUNK,), jnp.int32),
            pltpu.VMEM((IDX_PER_CHUNK,), jnp.int32),
            pltpu.VMEM((IDX_PER_CHUNK, D), jnp.float32),
            pltpu.VMEM((IDX_PER_CHUNK, D), jnp.float32),
            pltpu.VMEM((B, D), jnp.float32),
            pltpu.VMEM((B, D), jnp.float32),
            pltpu.SMEM((1,), jnp.int32),
            pltpu.SMEM((1,), jnp.int32),
            pltpu.VMEM((D,), jnp.float32),
            pltpu.SemaphoreType.DMA,
            pltpu.SemaphoreType.DMA,
            pltpu.SemaphoreType.DMA,
            pltpu.SemaphoreType.DMA,
        ],
    )(x, inds_flat, shadow)


def kernel(x, inds):
    inds_flat = inds.astype(jnp.int32).reshape(-1)
    shadow = _shadow_row(x).reshape(-1)
    return _pool(x, inds_flat, shadow)
